# Initial kernel scaffold; baseline (speedup 1.0000x reference)
import jax
import jax.numpy as jnp
from jax.experimental import pallas as pl


def kernel(x, adj_indices, adj_values, W, bias, gamma, beta):
    raise NotImplementedError("write your pallas kernel here")


# speedup vs baseline: 2.9750x; 2.9750x over previous
import functools

import jax
import jax.numpy as jnp
from jax import lax
from jax.experimental import pallas as pl
from jax.experimental.pallas import tpu as pltpu
from jax.experimental.pallas import tpu_sc as plsc

N = 10000
D = 128
NC = 2
NS = 16
NW = NC * NS
CH = 128
LANES = 16
N_PAD = 10240
ROWS_PER_TILE = N_PAD // NS
RB = 128

_BCAST_DNUMS = lax.GatherDimensionNumbers(
    offset_dims=(), collapsed_slice_dims=(0,), start_index_map=(0,))


def _bcast_lane(v, i):
    idx = jnp.full((LANES, 1), i, jnp.int32)
    return lax.gather(v, idx, _BCAST_DNUMS, (1,),
                      mode=lax.GatherScatterMode.PROMISE_IN_BOUNDS)


def _sc_aggregate(x, colp, rowp, valp, nchunks):
    mesh = plsc.VectorSubcoreMesh(core_axis_name="c", subcore_axis_name="s")

    def body(x_hbm, col_hbm, row_hbm, val_hbm, out_hbm,
             colv, rowv, valv, buf, acc):
        cid = lax.axis_index("c")
        sid = lax.axis_index("s")
        wid = sid * NC + cid

        pltpu.sync_copy(col_hbm.at[wid], colv)
        pltpu.sync_copy(row_hbm.at[wid], rowv)
        pltpu.sync_copy(val_hbm.at[wid], valv)

        zeros = jnp.zeros((LANES,), jnp.float32)

        def zrow(r, _):
            for k in range(D // LANES):
                buf[r, pl.ds(k * LANES, LANES)] = zeros
            return 0

        lax.fori_loop(0, RB, zrow, 0)
        base = sid * ROWS_PER_TILE
        for cblk in range(ROWS_PER_TILE // RB):
            pltpu.sync_copy(buf.at[pl.ds(0, RB)],
                            acc.at[pl.ds(base + cblk * RB, RB)])
        plsc.subcore_barrier()

        def chunk(j, _):
            pltpu.sync_copy(x_hbm.at[colv.at[j]], buf)

            def group(g, _):
                v16 = valv[j, pl.ds(g * LANES, LANES)]
                for i in range(LANES):
                    vb = _bcast_lane(v16, i)
                    e = g * LANES + i
                    for k in range(D // LANES):
                        sl = pl.ds(k * LANES, LANES)
                        buf[e, sl] = buf[e, sl] * vb
                return 0

            lax.fori_loop(0, CH // LANES, group, 0)
            pltpu.sync_copy(buf, acc.at[rowv.at[j]], add=True)
            return 0

        lax.fori_loop(0, nchunks, chunk, 0)
        plsc.subcore_barrier()

        sl = pl.ds(base, ROWS_PER_TILE)
        pltpu.sync_copy(acc.at[sl], out_hbm.at[cid, sl])

    run = pl.kernel(
        body,
        out_type=jax.ShapeDtypeStruct((NC, N_PAD, D), jnp.float32),
        mesh=mesh,
        scratch_types=[
            pltpu.VMEM((nchunks, CH), jnp.int32),
            pltpu.VMEM((nchunks, CH), jnp.int32),
            pltpu.VMEM((nchunks, CH), jnp.float32),
            pltpu.VMEM((CH, D), jnp.float32),
            pltpu.VMEM_SHARED((N_PAD, D), jnp.float32),
        ],
    )
    return run(x, colp, rowp, valp)


def _tc_body(p_ref, w_ref, b_ref, g_ref, be_ref, o_ref):
    y = p_ref[0] + p_ref[1]
    h2 = lax.dot_general(y, w_ref[...], (((1,), (1,)), ((), ())),
                         preferred_element_type=jnp.float32)
    h2 = h2 + b_ref[...]
    mean = jnp.mean(h2, axis=-1, keepdims=True)
    c = h2 - mean
    var = jnp.mean(c * c, axis=-1, keepdims=True)
    o_ref[...] = c * lax.rsqrt(var + 1e-5) * g_ref[...] + be_ref[...]


def _tc_epilogue(partials, W, bias, gamma, beta):
    BR = 1000
    grid = (N // BR,)
    return pl.pallas_call(
        _tc_body,
        grid=grid,
        in_specs=[
            pl.BlockSpec((NC, BR, D), lambda i: (0, i, 0)),
            pl.BlockSpec((D, D), lambda i: (0, 0)),
            pl.BlockSpec((1, D), lambda i: (0, 0)),
            pl.BlockSpec((1, D), lambda i: (0, 0)),
            pl.BlockSpec((1, D), lambda i: (0, 0)),
        ],
        out_specs=pl.BlockSpec((BR, D), lambda i: (i, 0)),
        out_shape=jax.ShapeDtypeStruct((N, D), jnp.float32),
    )(partials, W, bias.reshape(1, D), gamma.reshape(1, D),
      beta.reshape(1, D))


def kernel(x, adj_indices, adj_values, W, bias, gamma, beta):
    E = adj_values.shape[0]
    epw = -(-E // NW)
    nchunks = -(-epw // CH)
    nchunks = -(-nchunks // 8) * 8
    e_pad = NW * nchunks * CH

    row = adj_indices[0]
    col = adj_indices[1]
    pad = e_pad - E
    colp = jnp.pad(col, (0, pad)).reshape(NW, nchunks, CH)
    rowp = jnp.pad(row, (0, pad)).reshape(NW, nchunks, CH)
    valp = jnp.pad(adj_values, (0, pad)).reshape(NW, nchunks, CH)

    partials = _sc_aggregate(x, colp, rowp, valp, nchunks)
    return _tc_epilogue(partials, W, bias, gamma, beta)


# speedup vs baseline: 3.2509x; 1.0927x over previous
import jax
import jax.numpy as jnp
from jax import lax
from jax.experimental import pallas as pl
from jax.experimental.pallas import tpu as pltpu
from jax.experimental.pallas import tpu_sc as plsc

N = 10000
D = 128
NC = 2
NS = 16
NW = NC * NS
CH = 64
LANES = 16
NBUF = 4
MB = 8
N_PAD = 10240
ROWS_PER_TILE = N_PAD // NS

_BCAST_DNUMS = lax.GatherDimensionNumbers(
    offset_dims=(), collapsed_slice_dims=(0,), start_index_map=(0,))


def _bcast_lane(v, i):
    idx = jnp.full((LANES, 1), i, jnp.int32)
    return lax.gather(v, idx, _BCAST_DNUMS, (1,),
                      mode=lax.GatherScatterMode.PROMISE_IN_BOUNDS)


def _sc_aggregate(x, eidx, evals, nchunks):
    nblocks = nchunks // MB
    mesh = plsc.VectorSubcoreMesh(core_axis_name="c", subcore_axis_name="s")

    def body(x_hbm, ei_hbm, ev_hbm, out_hbm,
             idxb, valb, buf, gsem, isem, vsem, acc):
        cid = lax.axis_index("c")
        sid = lax.axis_index("s")
        wid = sid * NC + cid

        pltpu.sync_copy(ei_hbm.at[wid, 0], idxb.at[0])
        pltpu.sync_copy(ev_hbm.at[wid, 0], valb.at[0])
        pltpu.async_copy(ei_hbm.at[wid, 1], idxb.at[1], isem.at[1])
        pltpu.async_copy(ev_hbm.at[wid, 1], valb.at[1], vsem.at[1])

        zeros = jnp.zeros((LANES,), jnp.float32)

        def zrow(r, _):
            for k in range(D // LANES):
                buf[0, r, pl.ds(k * LANES, LANES)] = zeros
            return 0

        lax.fori_loop(0, CH, zrow, 0)
        base = sid * ROWS_PER_TILE
        for cblk in range(ROWS_PER_TILE // CH):
            pltpu.sync_copy(buf.at[0, pl.ds(0, CH)],
                            acc.at[pl.ds(base + cblk * CH, CH)])
        plsc.subcore_barrier()

        def gather_start(b, p, i):
            pltpu.async_copy(x_hbm.at[idxb.at[p, i * 2]], buf.at[b],
                             gsem.at[b])

        def scale(b, p, i):
            def group(g, _):
                v16 = valb[p, i, pl.ds(g * LANES, LANES)]
                for ii in range(LANES):
                    vb = _bcast_lane(v16, ii)
                    e = g * LANES + ii
                    for k in range(D // LANES):
                        sl = pl.ds(k * LANES, LANES)
                        buf[b, e, sl] = buf[b, e, sl] * vb
                return 0

            lax.fori_loop(0, CH // LANES, group, 0)

        gather_start(0, 0, 0)
        gather_start(1, 0, 1)

        def step(j, _):
            kb = j // MB
            i = lax.rem(j, MB)
            p = lax.rem(kb, 2)
            pn = lax.rem(kb + 1, 2)
            b = lax.rem(j, NBUF)
            br = lax.rem(j + 2, NBUF)

            pltpu.make_async_copy(x_hbm.at[pl.ds(0, CH)], buf.at[b],
                                  gsem.at[b]).wait()
            scale(b, p, i)
            pltpu.sync_copy(buf.at[b], acc.at[idxb.at[p, i * 2 + 1]],
                            add=True)

            @pl.when((i == 1) & (kb < nblocks - 1))
            def _():
                pltpu.async_copy(ei_hbm.at[wid, kb + 1], idxb.at[pn],
                                 isem.at[pn])
                pltpu.async_copy(ev_hbm.at[wid, kb + 1], valb.at[pn],
                                 vsem.at[pn])

            @pl.when((i == MB - 2) & (kb < nblocks - 1))
            def _():
                pltpu.make_async_copy(ei_hbm.at[wid, 0], idxb.at[pn],
                                      isem.at[pn]).wait()
                pltpu.make_async_copy(ev_hbm.at[wid, 0], valb.at[pn],
                                      vsem.at[pn]).wait()

            @pl.when(j + 2 < nchunks)
            def _():
                gather_start(br, lax.rem((j + 2) // MB, 2),
                             lax.rem(j + 2, MB))
            return 0

        lax.fori_loop(0, nchunks, step, 0)
        plsc.subcore_barrier()

        sl = pl.ds(base, ROWS_PER_TILE)
        pltpu.sync_copy(acc.at[sl], out_hbm.at[cid, sl])

        lax.fori_loop(0, CH, zrow, 0)
        for cblk in range(ROWS_PER_TILE // CH):
            pltpu.sync_copy(buf.at[0, pl.ds(0, CH)],
                            acc.at[pl.ds(base + cblk * CH, CH)])

    run = pl.kernel(
        body,
        out_type=jax.ShapeDtypeStruct((NC, N_PAD, D), jnp.float32),
        mesh=mesh,
        scratch_types=[
            pltpu.VMEM((2, MB * 2, CH), jnp.int32),
            pltpu.VMEM((2, MB, CH), jnp.float32),
            pltpu.VMEM((NBUF, CH, D), jnp.float32),
            pltpu.SemaphoreType.DMA((NBUF,)),
            pltpu.SemaphoreType.DMA((2,)),
            pltpu.SemaphoreType.DMA((2,)),
            pltpu.VMEM_SHARED((N_PAD, D), jnp.float32),
        ],
    )
    return run(x, eidx, evals)


def _tc_linear_body(x_ref, w_ref, h_ref):
    h_ref[...] = lax.dot_general(x_ref[...], w_ref[...],
                                 (((1,), (1,)), ((), ())),
                                 preferred_element_type=jnp.float32)


def _tc_linear(x, W):
    BR = 1000
    return pl.pallas_call(
        _tc_linear_body,
        grid=(N // BR,),
        in_specs=[
            pl.BlockSpec((BR, D), lambda i: (i, 0)),
            pl.BlockSpec((D, D), lambda i: (0, 0)),
        ],
        out_specs=pl.BlockSpec((BR, D), lambda i: (i, 0)),
        out_shape=jax.ShapeDtypeStruct((N, D), jnp.float32),
    )(x, W)


def _tc_body(p_ref, b_ref, g_ref, be_ref, o_ref):
    h2 = p_ref[0] + p_ref[1] + b_ref[...]
    mean = jnp.mean(h2, axis=-1, keepdims=True)
    c = h2 - mean
    var = jnp.mean(c * c, axis=-1, keepdims=True)
    o_ref[...] = c * lax.rsqrt(var + 1e-5) * g_ref[...] + be_ref[...]


def _tc_epilogue(partials, bias, gamma, beta):
    BR = 1000
    grid = (N // BR,)
    return pl.pallas_call(
        _tc_body,
        grid=grid,
        in_specs=[
            pl.BlockSpec((NC, BR, D), lambda i: (0, i, 0)),
            pl.BlockSpec((1, D), lambda i: (0, 0)),
            pl.BlockSpec((1, D), lambda i: (0, 0)),
            pl.BlockSpec((1, D), lambda i: (0, 0)),
        ],
        out_specs=pl.BlockSpec((BR, D), lambda i: (i, 0)),
        out_shape=jax.ShapeDtypeStruct((N, D), jnp.float32),
    )(partials, bias.reshape(1, D), gamma.reshape(1, D),
      beta.reshape(1, D))


def kernel(x, adj_indices, adj_values, W, bias, gamma, beta):
    E = adj_values.shape[0]
    epw = -(-E // NW)
    nchunks = -(-epw // CH)
    nchunks = -(-nchunks // MB) * MB
    e_pad = NW * nchunks * CH
    nblocks = nchunks // MB

    row = adj_indices[0]
    col = adj_indices[1]
    pad = e_pad - E
    colp = jnp.pad(col, (0, pad)).reshape(NW, nblocks, MB, 1, CH)
    rowp = jnp.pad(row, (0, pad)).reshape(NW, nblocks, MB, 1, CH)
    eidx = jnp.concatenate([colp, rowp], axis=3).reshape(NW, nblocks,
                                                         MB * 2, CH)
    evals = jnp.pad(adj_values, (0, pad)).reshape(NW, nblocks, MB, CH)

    h = _tc_linear(x, W)
    partials = _sc_aggregate(h, eidx, evals, nchunks)
    return _tc_epilogue(partials, bias, gamma, beta)
